# baseline (speedup 1.0000x reference)
import functools

import jax
import jax.numpy as jnp
from jax import lax
from jax.experimental import pallas as pl
from jax.experimental.pallas import tpu as pltpu
from jax.experimental.pallas import tpu_sc as plsc


def _make_sc_kernel(L, B, C, H, W, T):
    HW = H * W
    NW = 32
    WPL = NW // L
    TOK_W = HW // WPL
    NCHUNK = TOK_W // T
    CB = C // 16

    mesh = plsc.VectorSubcoreMesh(core_axis_name="c", subcore_axis_name="s")

    @functools.partial(
        pl.kernel,
        mesh=mesh,
        out_type=jax.ShapeDtypeStruct((L * HW, B * C), jnp.float32),
        scratch_types=[
            pltpu.VMEM((C, T), jnp.float32),
            pltpu.VMEM((T, C), jnp.float32),
            pltpu.VMEM((C,), jnp.float32),
        ],
        compiler_params=pltpu.CompilerParams(needs_layout_passes=False),
    )
    def k(feats_hbm, emb_hbm, out_hbm, in_v, out_v, emb_v):
        cid = lax.axis_index("c")
        sid = lax.axis_index("s")
        wid = sid * 2 + cid
        lvl = wid // WPL
        tok0 = (wid % WPL) * TOK_W
        pltpu.sync_copy(emb_hbm.at[pl.ds(lvl * C, C)], emb_v)
        embs = [emb_v[pl.ds(cb * 16, 16)] for cb in range(CB)]
        cidxs = [jnp.int32(cb * 16) + lax.iota(jnp.int32, 16) for cb in range(CB)]

        for b in range(B):
            rowbase_in = lvl * B * C + b * C

            def chunk_body(ci, _, b=b, rowbase_in=rowbase_in):
                tok = tok0 + ci * T
                pltpu.sync_copy(
                    feats_hbm.at[pl.ds(rowbase_in, C), pl.ds(tok, T)], in_v
                )

                def tok_body(t, _):
                    tsplat = jnp.full((16,), t, jnp.int32)
                    for cb in range(CB):
                        vec = plsc.load_gather(in_v, [cidxs[cb], tsplat])
                        out_v[t, pl.ds(cb * 16, 16)] = vec + embs[cb]
                    return 0

                lax.fori_loop(0, T, tok_body, 0)
                pltpu.sync_copy(
                    out_v,
                    out_hbm.at[pl.ds(lvl * HW + tok, T), pl.ds(b * C, C)],
                )
                return 0

            lax.fori_loop(0, NCHUNK, chunk_body, 0)

    return k


def _tc_body(feats_ref, embed_ref, out_ref):
    B = feats_ref.shape[1]
    C = feats_ref.shape[2]
    hb = feats_ref.shape[3]
    W = feats_ref.shape[4]
    e = embed_ref[0]
    for b in range(B):
        x = feats_ref[0, b].reshape(C, hb * W)
        out_ref[:, b, :] = x.T + e


def _tc_kernel(feats, embed_weight):
    L, B, C, H, W = feats.shape
    hb = 8
    n_hblk = H // hb
    return pl.pallas_call(
        _tc_body,
        grid=(L, n_hblk),
        in_specs=[
            pl.BlockSpec((1, B, C, hb, W), lambda l, j: (l, 0, 0, j, 0)),
            pl.BlockSpec((1, 1, C), lambda l, j: (l, 0, 0)),
        ],
        out_specs=pl.BlockSpec(
            (hb * W, B, C), lambda l, j: (l * (H // hb) + j, 0, 0)
        ),
        out_shape=jax.ShapeDtypeStruct((L * H * W, B, C), feats.dtype),
    )(feats, embed_weight.reshape(L, 1, C))


def kernel(feats, level_start_idx, spatial_shapes, embed_weight):
    L, B, C, H, W = feats.shape
    k = _make_sc_kernel(L, B, C, H, W, T=128)
    out2d = k(
        feats.reshape(L * B * C, H * W),
        embed_weight.reshape(L * C),
    )
    return out2d.reshape(L * H * W, B, C)


# speedup vs baseline: 1.0467x; 1.0467x over previous
import functools

import jax
import jax.numpy as jnp
from jax import lax
from jax.experimental import pallas as pl
from jax.experimental.pallas import tpu as pltpu
from jax.experimental.pallas import tpu_sc as plsc


def _make_sc_kernel(L, B, C, H, W, T):
    HW = H * W
    NW = 32
    WPL = NW // L
    TOK_W = HW // WPL
    NCHUNK = TOK_W // T
    CB = C // 16

    mesh = plsc.VectorSubcoreMesh(core_axis_name="c", subcore_axis_name="s")

    @functools.partial(
        pl.kernel,
        mesh=mesh,
        out_type=jax.ShapeDtypeStruct((L * HW, B * C), jnp.float32),
        scratch_types=[
            pltpu.VMEM((C, T), jnp.float32),
            pltpu.VMEM((C, T), jnp.float32),
            pltpu.VMEM((T, C), jnp.float32),
            pltpu.VMEM((C,), jnp.float32),
            pltpu.SemaphoreType.DMA,
            pltpu.SemaphoreType.DMA,
            pltpu.SemaphoreType.DMA,
        ],
        compiler_params=pltpu.CompilerParams(needs_layout_passes=False),
    )
    def k(feats_hbm, emb_hbm, out_hbm, in_v0, in_v1, out_v0, emb_v,
          is0, is1, os0):
        cid = lax.axis_index("c")
        sid = lax.axis_index("s")
        wid = sid * 2 + cid
        lvl = wid // WPL
        tok0 = (wid % WPL) * TOK_W
        pltpu.sync_copy(emb_hbm.at[pl.ds(lvl * C, C)], emb_v)
        embs = [emb_v[pl.ds(cb * 16, 16)] for cb in range(CB)]
        cidxs = [jnp.int32(cb * 16) + lax.iota(jnp.int32, 16) for cb in range(CB)]

        in_bufs = (in_v0, in_v1)
        in_sems = (is0, is1)
        units = [(b, ci) for b in range(B) for ci in range(NCHUNK)]
        NSPLIT = 4
        CS = C // NSPLIT

        def start_in(g):
            b, ci = units[g]
            tok = tok0 + ci * T
            buf = in_bufs[g % 2]
            sem = in_sems[g % 2]
            return [
                pltpu.async_copy(
                    feats_hbm.at[
                        pl.ds(lvl * B * C + b * C + q * CS, CS), pl.ds(tok, T)
                    ],
                    buf.at[pl.ds(q * CS, CS), :],
                    sem,
                )
                for q in range(NSPLIT)
            ]

        def out_slice(b, ci):
            tok = tok0 + ci * T
            return out_hbm.at[pl.ds(lvl * HW + tok, T), pl.ds(b * C, C)]

        def compute(in_v, out_v):
            def tok_body(t, _):
                tsplat = jnp.full((16,), t, jnp.int32)
                for cb in range(CB):
                    vec = plsc.load_gather(in_v, [cidxs[cb], tsplat])
                    out_v[t, pl.ds(cb * 16, 16)] = vec + embs[cb]
                return 0

            lax.fori_loop(0, T, tok_body, 0)

        n = len(units)
        h_in = {}
        h_out = {}
        h_in[0] = start_in(0)
        for g in range(n):
            s = g % 2
            if g + 1 < n:
                h_in[g + 1] = start_in(g + 1)
            for h in h_in.pop(g):
                h.wait()
            if g - 1 >= 0:
                h_out[g - 1].wait()
            compute(in_bufs[s], out_v0)
            h_out[g] = pltpu.async_copy(out_v0, out_slice(*units[g]), os0)
        h_out[n - 1].wait()

    return k


def _tc_body(feats_ref, embed_ref, out_ref):
    B = feats_ref.shape[1]
    C = feats_ref.shape[2]
    hb = feats_ref.shape[3]
    W = feats_ref.shape[4]
    e = embed_ref[0]
    for b in range(B):
        x = feats_ref[0, b].reshape(C, hb * W)
        out_ref[:, b, :] = x.T + e


def _tc_kernel(feats, embed_weight):
    L, B, C, H, W = feats.shape
    hb = 8
    n_hblk = H // hb
    return pl.pallas_call(
        _tc_body,
        grid=(L, n_hblk),
        in_specs=[
            pl.BlockSpec((1, B, C, hb, W), lambda l, j: (l, 0, 0, j, 0)),
            pl.BlockSpec((1, 1, C), lambda l, j: (l, 0, 0)),
        ],
        out_specs=pl.BlockSpec(
            (hb * W, B, C), lambda l, j: (l * (H // hb) + j, 0, 0)
        ),
        out_shape=jax.ShapeDtypeStruct((L * H * W, B, C), feats.dtype),
    )(feats, embed_weight.reshape(L, 1, C))


def kernel(feats, level_start_idx, spatial_shapes, embed_weight):
    L, B, C, H, W = feats.shape
    k = _make_sc_kernel(L, B, C, H, W, T=128)
    out2d = k(
        feats.reshape(L * B * C, H * W),
        embed_weight.reshape(L * C),
    )
    return out2d.reshape(L * H * W, B, C)
